# Initial kernel scaffold; baseline (speedup 1.0000x reference)
import jax
import jax.numpy as jnp
from jax.experimental import pallas as pl


def kernel(X, conv_w0, conv_b0, final_w0, final_b0, router_w0, router_b0, keys0, conv_w1, conv_b1, final_w1, final_b1, router_w1, router_b1, keys1, conv_w2, conv_b2, final_w2, final_b2, router_w2, router_b2, keys2, lin_w, lin_b):
    raise NotImplementedError("write your pallas kernel here")


# speedup vs baseline: 2.5137x; 2.5137x over previous
import functools

import jax
import jax.numpy as jnp
import numpy as np
from jax.experimental import pallas as pl

_FF = 4
_LAYERS = [dict(cin=3, cout=8, ps=16), dict(cin=8, cout=16, ps=13), dict(cin=16, cout=16, ps=10)]
_E = 8
_NPATCH = {0: 16, 1: 17, 2: 32}


def _fourier_feats(H, W):
    ys = (jnp.arange(H, dtype=jnp.float32) + 0.5) / H * 2.0 - 1.0
    xs = (jnp.arange(W, dtype=jnp.float32) + 0.5) / W * 2.0 - 1.0
    yy, xx = jnp.meshgrid(ys, xs, indexing='ij')
    feats = [xx, yy, xx * yy, xx ** 2 + yy ** 2]
    for f in range(_FF):
        s = (2.0 ** f) * jnp.pi
        for g in (xx, yy, xx + yy, xx - yy):
            feats.append(jnp.sin(s * g))
            feats.append(jnp.cos(s * g))
    return jnp.stack(feats, axis=0)


def _moe_kernel(ps, cinp, cout, x_ref, fb_ref, mx_ref, wc_ref, bc_ref, fwb_ref, fbb_ref, o_ref):
    x = x_ref[...]
    npat = x.shape[0]
    mean_x = jnp.mean(x, axis=(1, 2))
    logits = jnp.dot(mean_x, mx_ref[...], preferred_element_type=jnp.float32)
    logits = logits + fb_ref[...].reshape(npat, _E)
    mmax = jnp.max(logits, axis=-1, keepdims=True)
    ex = jnp.exp(logits - mmax)
    scores = ex / jnp.sum(ex, axis=-1, keepdims=True)
    lane = jax.lax.broadcasted_iota(jnp.int32, (_E, _E * cout), 1) // cout
    row = jax.lax.broadcasted_iota(jnp.int32, (_E, _E * cout), 0)
    sel = (lane == row).astype(jnp.float32)
    sexp = jnp.dot(scores, sel, preferred_element_type=jnp.float32)
    zy = jnp.zeros((npat, 1, ps, cinp), jnp.float32)
    xp = jnp.concatenate([zy, x, zy], axis=1)
    zx = jnp.zeros((npat, ps + 2, 1, cinp), jnp.float32)
    xp = jnp.concatenate([zx, xp, zx], axis=2)
    cols = [xp[:, dy:dy + ps, dx:dx + ps, :] for dy in range(3) for dx in range(3)]
    a = jnp.concatenate(cols, axis=3).reshape(npat * ps * ps, 9 * cinp)
    z = jnp.dot(a, wc_ref[...], preferred_element_type=jnp.float32) + bc_ref[...]
    z = jnp.maximum(z, 0.0).reshape(npat, ps * ps, _E * cout)
    zw = (z * sexp[:, None, :]).reshape(npat * ps * ps, _E * cout)
    out = jnp.dot(zw, fwb_ref[...], preferred_element_type=jnp.float32) + fbb_ref[...]
    o_ref[...] = out.reshape(npat, ps, ps, cout)


def _moe_layer(X, l, conv_w, conv_b, final_w, final_b, router_w, router_b, keys):
    cfg = _LAYERS[l]
    ps, cin, cout = cfg['ps'], cfg['cin'], cfg['cout']
    B, H, W = X.shape[0], X.shape[1], X.shape[2]
    hp, wp = H // ps, W // ps
    P = hp * wp
    Xc = X[:, :hp * ps, :wp * ps, :]
    Xp = Xc.reshape(B, hp, ps, wp, ps, cin).transpose(0, 1, 3, 2, 4, 5)
    Xp = Xp.reshape(B * P, ps, ps, cin)
    cinp = 8 * ((cin + 7) // 8)
    if cinp != cin:
        Xp = jnp.pad(Xp, ((0, 0), (0, 0), (0, 0), (0, cinp - cin)))
    fc = 4 + 8 * _FF
    Wr = router_w[:, :, 0, 0]
    M = Wr.T @ keys.T
    cvec = router_b @ keys.T
    four = _fourier_feats(hp * ps, wp * ps)
    fmean = four.reshape(fc, hp, ps, wp, ps).mean(axis=(2, 4))
    fmean = fmean.transpose(1, 2, 0).reshape(P, fc)
    fbias = fmean @ M[cin:] + cvec
    fbias = jnp.tile(fbias, (B, 1))
    Mx = M[:cin]
    if cinp != cin:
        Mx = jnp.pad(Mx, ((0, cinp - cin), (0, 0)))
    Wc = conv_w.transpose(3, 4, 2, 0, 1)
    if cinp != cin:
        Wc = jnp.pad(Wc, ((0, 0), (0, 0), (0, cinp - cin), (0, 0), (0, 0)))
    Wc = Wc.reshape(9 * cinp, _E * cout)
    bc = conv_b.reshape(1, _E * cout)
    FwB = jnp.tile(final_w[:, :, 0, 0].T, (_E, 1))
    fbb = final_b.reshape(1, cout)
    npat = _NPATCH[l]
    G = (B * P) // npat
    fbias = fbias.reshape(G, npat, _E)
    kfn = functools.partial(_moe_kernel, ps, cinp, cout)
    out = pl.pallas_call(
        kfn,
        grid=(G,),
        in_specs=[
            pl.BlockSpec((npat, ps, ps, cinp), lambda i: (i, 0, 0, 0)),
            pl.BlockSpec((1, npat, _E), lambda i: (i, 0, 0)),
            pl.BlockSpec((cinp, _E), lambda i: (0, 0)),
            pl.BlockSpec((9 * cinp, _E * cout), lambda i: (0, 0)),
            pl.BlockSpec((1, _E * cout), lambda i: (0, 0)),
            pl.BlockSpec((_E * cout, cout), lambda i: (0, 0)),
            pl.BlockSpec((1, cout), lambda i: (0, 0)),
        ],
        out_specs=pl.BlockSpec((npat, ps, ps, cout), lambda i: (i, 0, 0, 0)),
        out_shape=jax.ShapeDtypeStruct((B * P, ps, ps, cout), jnp.float32),
    )(Xp, fbias, Mx, Wc, bc, FwB, fbb)
    img = out.reshape(B, hp, wp, ps, ps, cout).transpose(0, 1, 3, 2, 4, 5)
    return img.reshape(B, hp * ps, wp * ps, cout)


def _pool_kernel(H, W, C, x_ref, ph_ref, pw_ref, o_ref):
    x = x_ref[...].reshape(H, W * C)
    t = jnp.dot(ph_ref[...], x, preferred_element_type=jnp.float32)
    t = t.reshape(8, W, C)
    s = jax.lax.dot_general(t, pw_ref[...], (((1,), (0,)), ((), ())),
                            preferred_element_type=jnp.float32)
    o_ref[...] = s[None]


def _cls_kernel(x_ref, lw_ref, lb_ref, o_ref):
    out = jax.lax.dot_general(x_ref[...], lw_ref[...], (((1,), (1,)), ((), ())),
                              preferred_element_type=jnp.float32)
    o_ref[...] = out + lb_ref[...]


def _pool_mat(n, out=8):
    m = np.zeros((n, out), dtype=np.float32)
    for i in range(out):
        h0 = (i * n) // out
        h1 = -((-(i + 1) * n) // out)
        m[h0:h1, i] = 1.0 / (h1 - h0)
    return jnp.asarray(m)


def kernel(X, conv_w0, conv_b0, final_w0, final_b0, router_w0, router_b0, keys0,
           conv_w1, conv_b1, final_w1, final_b1, router_w1, router_b1, keys1,
           conv_w2, conv_b2, final_w2, final_b2, router_w2, router_b2, keys2,
           lin_w, lin_b):
    X = X.transpose(0, 2, 3, 1)
    X = _moe_layer(X, 0, conv_w0, conv_b0, final_w0, final_b0, router_w0, router_b0, keys0)
    X = _moe_layer(X, 1, conv_w1, conv_b1, final_w1, final_b1, router_w1, router_b1, keys1)
    X = _moe_layer(X, 2, conv_w2, conv_b2, final_w2, final_b2, router_w2, router_b2, keys2)
    B, H, W, C = X.shape
    ncls = lin_w.shape[0]
    ph = _pool_mat(H).T
    pw = _pool_mat(W)
    lwp = lin_w.reshape(ncls, C, 8, 8).transpose(0, 2, 1, 3).reshape(ncls, C * 64)
    pooled = pl.pallas_call(
        functools.partial(_pool_kernel, H, W, C),
        grid=(B,),
        in_specs=[
            pl.BlockSpec((1, H, W, C), lambda b: (b, 0, 0, 0)),
            pl.BlockSpec((8, H), lambda b: (0, 0)),
            pl.BlockSpec((W, 8), lambda b: (0, 0)),
        ],
        out_specs=pl.BlockSpec((1, 8, C, 8), lambda b: (b, 0, 0, 0)),
        out_shape=jax.ShapeDtypeStruct((B, 8, C, 8), jnp.float32),
    )(X, ph, pw)
    flat = pooled.reshape(B, C * 64)
    out = pl.pallas_call(
        _cls_kernel,
        in_specs=[
            pl.BlockSpec((B, C * 64), lambda: (0, 0)),
            pl.BlockSpec((ncls, C * 64), lambda: (0, 0)),
            pl.BlockSpec((1, ncls), lambda: (0, 0)),
        ],
        out_specs=pl.BlockSpec((B, ncls), lambda: (0, 0)),
        out_shape=jax.ShapeDtypeStruct((B, ncls), jnp.float32),
    )(flat, lwp, lin_b.reshape(1, ncls))
    return out


# speedup vs baseline: 5.6374x; 2.2426x over previous
import functools

import jax
import jax.numpy as jnp
import numpy as np
from jax.experimental import pallas as pl

_FF = 4
_LAYERS = [dict(cin=3, cout=8, ps=16), dict(cin=8, cout=16, ps=13), dict(cin=16, cout=16, ps=10)]
_E = 8


def _fourier_patch_means(H, W, hp, wp, ps):
    ys = ((np.arange(H, dtype=np.float32) + 0.5) / np.float32(H) * np.float32(2.0)
          - np.float32(1.0)).astype(np.float32)
    xs = ((np.arange(W, dtype=np.float32) + 0.5) / np.float32(W) * np.float32(2.0)
          - np.float32(1.0)).astype(np.float32)
    yy, xx = np.meshgrid(ys, xs, indexing='ij')
    feats = [xx, yy, (xx * yy).astype(np.float32),
             (xx ** 2 + yy ** 2).astype(np.float32)]
    for f in range(_FF):
        s = np.float32((2.0 ** f) * np.pi)
        for g in (xx, yy, (xx + yy).astype(np.float32), (xx - yy).astype(np.float32)):
            feats.append(np.sin(s * g, dtype=np.float32))
            feats.append(np.cos(s * g, dtype=np.float32))
    four = np.stack(feats, axis=0).astype(np.float32)
    fc = four.shape[0]
    fmean = four.reshape(fc, hp, ps, wp, ps).mean(axis=(2, 4), dtype=np.float32)
    return fmean.transpose(1, 2, 0).reshape(hp * wp, fc).astype(np.float32)


def _moe_kernel(ps, Win, Wv, C, cout, wp,
                x_ref, fb_ref, mx_ref, wc_ref, bc_ref, fwb_ref, fbb_ref,
                mneg_ref, mpos_ref, rep_ref, o_ref):
    EC = _E * cout
    x = x_ref[...].reshape(ps, C, Win)
    if Win != Wv:
        x = x[:, :, :Wv]
    xm = jnp.mean(x, axis=0)
    pm = jnp.dot(xm, rep_ref[...], preferred_element_type=jnp.float32) * (1.0 / ps)
    logits = jax.lax.dot_general(pm, mx_ref[...], (((0,), (0,)), ((), ())),
                                 preferred_element_type=jnp.float32)
    logits = logits + fb_ref[...].reshape(wp, _E)
    mmax = jnp.max(logits, axis=-1, keepdims=True)
    ex = jnp.exp(logits - mmax)
    scores = ex / jnp.sum(ex, axis=-1, keepdims=True)
    lane = jax.lax.broadcasted_iota(jnp.int32, (_E, EC), 1) // cout
    row = jax.lax.broadcasted_iota(jnp.int32, (_E, EC), 0)
    sel = (lane == row).astype(jnp.float32)
    sexp = jnp.dot(scores, sel, preferred_element_type=jnp.float32)
    sexp_w = jax.lax.dot_general(rep_ref[...], sexp, (((1,), (0,)), ((), ())),
                                 preferred_element_type=jnp.float32)
    z1 = jnp.zeros((ps, C, 1), jnp.float32)
    xm1 = jnp.concatenate([z1, x[:, :, :Wv - 1]], axis=2) * mneg_ref[...]
    xp1 = jnp.concatenate([x[:, :, 1:], z1], axis=2) * mpos_ref[...]
    zrow = jnp.zeros((1, C, Wv), jnp.float32)
    pads = [jnp.concatenate([zrow, s, zrow], axis=0) for s in (xm1, x, xp1)]
    z = jnp.zeros((ps, Wv, EC), jnp.float32) + bc_ref[...][None]
    for dy in range(3):
        for dx in range(3):
            opnd = pads[dx][dy:dy + ps]
            z = z + jax.lax.dot_general(opnd, wc_ref[3 * dy + dx],
                                        (((1,), (0,)), ((), ())),
                                        preferred_element_type=jnp.float32)
    z = jnp.maximum(z, 0.0) * sexp_w[None]
    t = jax.lax.dot_general(fwb_ref[...], z, (((0,), (2,)), ((), ())),
                            preferred_element_type=jnp.float32)
    t = t + fbb_ref[...].reshape(cout, 1, 1)
    o_ref[...] = t.transpose(1, 0, 2)[None]


def _moe_layer(X, l, conv_w, conv_b, final_w, final_b, router_w, router_b, keys):
    cfg = _LAYERS[l]
    ps, cin, cout = cfg['ps'], cfg['cin'], cfg['cout']
    B, Hin, C, Win = X.shape
    hp, wp = Hin // ps, Win // ps
    Wv = wp * ps
    P = hp * wp
    EC = _E * cout
    Wr = router_w[:, :, 0, 0]
    M = Wr.T @ keys.T
    cvec = router_b @ keys.T
    fmean = jnp.asarray(_fourier_patch_means(hp * ps, Wv, hp, wp, ps))
    fbias = (fmean @ M[cin:] + cvec).reshape(hp, wp, _E)
    Mx = M[:cin]
    if C != cin:
        Mx = jnp.pad(Mx, ((0, C - cin), (0, 0)))
    Wc = conv_w.transpose(3, 4, 2, 0, 1)
    if C != cin:
        Wc = jnp.pad(Wc, ((0, 0), (0, 0), (0, C - cin), (0, 0), (0, 0)))
    Wc = Wc.reshape(9, C, EC)
    bc = conv_b.reshape(1, EC)
    FwB = jnp.tile(final_w[:, :, 0, 0].T, (_E, 1))
    fbb = final_b.reshape(1, cout)
    wl = np.arange(Wv) % ps
    mneg = jnp.asarray((wl != 0).astype(np.float32).reshape(1, 1, Wv))
    mpos = jnp.asarray((wl != ps - 1).astype(np.float32).reshape(1, 1, Wv))
    rep_np = np.zeros((Wv, wp), dtype=np.float32)
    rep_np[np.arange(Wv), np.arange(Wv) // ps] = 1.0
    rep = jnp.asarray(rep_np)
    kfn = functools.partial(_moe_kernel, ps, Win, Wv, C, cout, wp)
    out = pl.pallas_call(
        kfn,
        grid=(B, hp),
        in_specs=[
            pl.BlockSpec((1, ps, C, Win), lambda b, i: (b, i, 0, 0)),
            pl.BlockSpec((1, wp, _E), lambda b, i: (i, 0, 0)),
            pl.BlockSpec((C, _E), lambda b, i: (0, 0)),
            pl.BlockSpec((9, C, EC), lambda b, i: (0, 0, 0)),
            pl.BlockSpec((1, EC), lambda b, i: (0, 0)),
            pl.BlockSpec((EC, cout), lambda b, i: (0, 0)),
            pl.BlockSpec((1, cout), lambda b, i: (0, 0)),
            pl.BlockSpec((1, 1, Wv), lambda b, i: (0, 0, 0)),
            pl.BlockSpec((1, 1, Wv), lambda b, i: (0, 0, 0)),
            pl.BlockSpec((Wv, wp), lambda b, i: (0, 0)),
        ],
        out_specs=pl.BlockSpec((1, ps, cout, Wv), lambda b, i: (b, i, 0, 0)),
        out_shape=jax.ShapeDtypeStruct((B, hp * ps, cout, Wv), jnp.float32),
    )(X, fbias, Mx, Wc, bc, FwB, fbb, mneg, mpos, rep)
    return out


def _pool_kernel(H, W, C, x_ref, ph_ref, q_ref, o_ref):
    x = x_ref[...].reshape(H, C * W)
    t = jnp.dot(ph_ref[...], x, preferred_element_type=jnp.float32)
    s = jnp.dot(t, q_ref[...], preferred_element_type=jnp.float32)
    o_ref[...] = s[None]


def _cls_kernel(x_ref, lw_ref, lb_ref, o_ref):
    out = jax.lax.dot_general(x_ref[...], lw_ref[...], (((1,), (1,)), ((), ())),
                              preferred_element_type=jnp.float32)
    o_ref[...] = out + lb_ref[...]


def _pool_mat(n, out=8):
    m = np.zeros((n, out), dtype=np.float32)
    for i in range(out):
        h0 = (i * n) // out
        h1 = -((-(i + 1) * n) // out)
        m[h0:h1, i] = 1.0 / (h1 - h0)
    return m


def kernel(X, conv_w0, conv_b0, final_w0, final_b0, router_w0, router_b0, keys0,
           conv_w1, conv_b1, final_w1, final_b1, router_w1, router_b1, keys1,
           conv_w2, conv_b2, final_w2, final_b2, router_w2, router_b2, keys2,
           lin_w, lin_b):
    X = X.transpose(0, 2, 1, 3)
    X = jnp.pad(X, ((0, 0), (0, 0), (0, 5), (0, 0)))
    X = _moe_layer(X, 0, conv_w0, conv_b0, final_w0, final_b0, router_w0, router_b0, keys0)
    X = _moe_layer(X, 1, conv_w1, conv_b1, final_w1, final_b1, router_w1, router_b1, keys1)
    X = _moe_layer(X, 2, conv_w2, conv_b2, final_w2, final_b2, router_w2, router_b2, keys2)
    B, H, C, W = X.shape
    ncls = lin_w.shape[0]
    ph = jnp.asarray(_pool_mat(H).T)
    pw = _pool_mat(W)
    q_np = np.einsum('cd,wj->cwdj', np.eye(C, dtype=np.float32), pw).reshape(C * W, C * 8)
    q = jnp.asarray(q_np)
    lwp = lin_w.reshape(ncls, C, 8, 8).transpose(0, 2, 1, 3).reshape(ncls, C * 64)
    pooled = pl.pallas_call(
        functools.partial(_pool_kernel, H, W, C),
        grid=(B,),
        in_specs=[
            pl.BlockSpec((1, H, C, W), lambda b: (b, 0, 0, 0)),
            pl.BlockSpec((8, H), lambda b: (0, 0)),
            pl.BlockSpec((C * W, C * 8), lambda b: (0, 0)),
        ],
        out_specs=pl.BlockSpec((1, 8, C * 8), lambda b: (b, 0, 0)),
        out_shape=jax.ShapeDtypeStruct((B, 8, C * 8), jnp.float32),
    )(X, ph, q)
    flat = pooled.reshape(B, C * 64)
    out = pl.pallas_call(
        _cls_kernel,
        in_specs=[
            pl.BlockSpec((B, C * 64), lambda: (0, 0)),
            pl.BlockSpec((ncls, C * 64), lambda: (0, 0)),
            pl.BlockSpec((1, ncls), lambda: (0, 0)),
        ],
        out_specs=pl.BlockSpec((B, ncls), lambda: (0, 0)),
        out_shape=jax.ShapeDtypeStruct((B, ncls), jnp.float32),
    )(flat, lwp, lin_b.reshape(1, ncls))
    return out


# speedup vs baseline: 11.9017x; 2.1112x over previous
import functools

import jax
import jax.numpy as jnp
import numpy as np
from jax.experimental import pallas as pl

_FF = 4
_LAYERS = [dict(cin=3, cout=8, ps=16), dict(cin=8, cout=16, ps=13), dict(cin=16, cout=16, ps=10)]
_E = 8


def _fourier_patch_means(H, W, hp, wp, ps):
    ys = ((np.arange(H, dtype=np.float32) + 0.5) / np.float32(H) * np.float32(2.0)
          - np.float32(1.0)).astype(np.float32)
    xs = ((np.arange(W, dtype=np.float32) + 0.5) / np.float32(W) * np.float32(2.0)
          - np.float32(1.0)).astype(np.float32)
    yy, xx = np.meshgrid(ys, xs, indexing='ij')
    feats = [xx, yy, (xx * yy).astype(np.float32),
             (xx.astype(np.float32) ** 2 + yy.astype(np.float32) ** 2).astype(np.float32)]
    for f in range(_FF):
        s = np.float32((2.0 ** f) * np.pi)
        for g in (xx, yy, (xx + yy).astype(np.float32), (xx - yy).astype(np.float32)):
            feats.append(np.sin(s * g, dtype=np.float32))
            feats.append(np.cos(s * g, dtype=np.float32))
    four = np.stack(feats, axis=0).astype(np.float32)
    fc = four.shape[0]
    fmean = four.reshape(fc, hp, ps, wp, ps).mean(axis=(2, 4), dtype=np.float32)
    return fmean.transpose(1, 2, 0).reshape(hp * wp, fc).astype(np.float32)


def _moe_kernel(ps, Win, Wv, C, cout, wp,
                x_ref, fb_ref, mx_ref, wc_ref, bc_ref, fwb_ref, fbb_ref,
                mneg_ref, mpos_ref, rep_ref, rept_ref, o_ref):
    EC = _E * cout
    x = x_ref[...].reshape(ps, Win, C)
    if Win != Wv:
        x = x[:, :Wv, :]
    xm = jnp.mean(x, axis=0)
    pm = jnp.dot(rept_ref[...], xm, preferred_element_type=jnp.float32) * (1.0 / ps)
    logits = jnp.dot(pm, mx_ref[...], preferred_element_type=jnp.float32)
    logits = logits + fb_ref[...].reshape(wp, _E)
    mmax = jnp.max(logits, axis=-1, keepdims=True)
    ex = jnp.exp(logits - mmax)
    scores = ex / jnp.sum(ex, axis=-1, keepdims=True)
    lane = jax.lax.broadcasted_iota(jnp.int32, (_E, EC), 1) // cout
    row = jax.lax.broadcasted_iota(jnp.int32, (_E, EC), 0)
    sel = (lane == row).astype(jnp.float32)
    sexp = jnp.dot(scores, sel, preferred_element_type=jnp.float32)
    sexp_w = jnp.dot(rep_ref[...], sexp, preferred_element_type=jnp.float32)
    z1 = jnp.zeros((ps, 1, C), jnp.float32)
    xm1 = jnp.concatenate([z1, x[:, :Wv - 1, :]], axis=1) * mneg_ref[...]
    xp1 = jnp.concatenate([x[:, 1:, :], z1], axis=1) * mpos_ref[...]
    a2 = jnp.concatenate([xm1, x, xp1], axis=2)
    zrow = jnp.zeros((1, Wv, 3 * C), jnp.float32)
    a2p = jnp.concatenate([zrow, a2, zrow], axis=0)
    z = jnp.zeros((ps * Wv, EC), jnp.float32) + bc_ref[...]
    for dy in range(3):
        opnd = a2p[dy:dy + ps].reshape(ps * Wv, 3 * C)
        z = z + jnp.dot(opnd, wc_ref[dy],
                        preferred_element_type=jnp.float32)
    z = jnp.maximum(z, 0.0).reshape(ps, Wv, EC) * sexp_w[None]
    out = jnp.dot(z.reshape(ps * Wv, EC), fwb_ref[...],
                  preferred_element_type=jnp.float32) + fbb_ref[...]
    o_ref[...] = out.reshape(1, ps, Wv, cout)


def _moe_layer(X, l, conv_w, conv_b, final_w, final_b, router_w, router_b, keys):
    cfg = _LAYERS[l]
    ps, cin, cout = cfg['ps'], cfg['cin'], cfg['cout']
    B, Hin, Win, C = X.shape
    hp, wp = Hin // ps, Win // ps
    Wv = wp * ps
    P = hp * wp
    EC = _E * cout
    fc = 4 + 8 * _FF
    Wr = router_w[:, :, 0, 0]
    M = Wr.T @ keys.T
    cvec = router_b @ keys.T
    fmean = jnp.asarray(_fourier_patch_means(hp * ps, Wv, hp, wp, ps))
    fbias = (fmean @ M[cin:] + cvec).reshape(hp, wp, _E)
    Mx = M[:cin]
    if C != cin:
        Mx = jnp.pad(Mx, ((0, C - cin), (0, 0)))
    Wc = conv_w.transpose(3, 4, 2, 0, 1)
    if C != cin:
        Wc = jnp.pad(Wc, ((0, 0), (0, 0), (0, C - cin), (0, 0), (0, 0)))
    Wc = Wc.reshape(3, 3 * C, EC)
    bc = conv_b.reshape(1, EC)
    FwB = jnp.tile(final_w[:, :, 0, 0].T, (_E, 1))
    fbb = final_b.reshape(1, cout)
    wl = np.arange(Wv) % ps
    mneg = jnp.asarray((wl != 0).astype(np.float32).reshape(1, Wv, 1))
    mpos = jnp.asarray((wl != ps - 1).astype(np.float32).reshape(1, Wv, 1))
    rep_np = np.zeros((Wv, wp), dtype=np.float32)
    rep_np[np.arange(Wv), np.arange(Wv) // ps] = 1.0
    rep = jnp.asarray(rep_np)
    rept = jnp.asarray(rep_np.T)
    kfn = functools.partial(_moe_kernel, ps, Win, Wv, C, cout, wp)
    out = pl.pallas_call(
        kfn,
        grid=(B, hp),
        in_specs=[
            pl.BlockSpec((1, ps, Win, C), lambda b, i: (b, i, 0, 0)),
            pl.BlockSpec((1, wp, _E), lambda b, i: (i, 0, 0)),
            pl.BlockSpec((C, _E), lambda b, i: (0, 0)),
            pl.BlockSpec((3, 3 * C, EC), lambda b, i: (0, 0, 0)),
            pl.BlockSpec((1, EC), lambda b, i: (0, 0)),
            pl.BlockSpec((EC, cout), lambda b, i: (0, 0)),
            pl.BlockSpec((1, cout), lambda b, i: (0, 0)),
            pl.BlockSpec((1, Wv, 1), lambda b, i: (0, 0, 0)),
            pl.BlockSpec((1, Wv, 1), lambda b, i: (0, 0, 0)),
            pl.BlockSpec((Wv, wp), lambda b, i: (0, 0)),
            pl.BlockSpec((wp, Wv), lambda b, i: (0, 0)),
        ],
        out_specs=pl.BlockSpec((1, ps, Wv, cout), lambda b, i: (b, i, 0, 0)),
        out_shape=jax.ShapeDtypeStruct((B, hp * ps, Wv, cout), jnp.float32),
    )(X, fbias, Mx, Wc, bc, FwB, fbb, mneg, mpos, rep, rept)
    return out


def _pool_kernel(H, W, C, x_ref, ph_ref, q_ref, o_ref):
    x = x_ref[...].reshape(H, W * C)
    t = jnp.dot(ph_ref[...], x, preferred_element_type=jnp.float32)
    s = jnp.dot(t, q_ref[...], preferred_element_type=jnp.float32)
    o_ref[...] = s[None]


def _cls_kernel(x_ref, lw_ref, lb_ref, o_ref):
    out = jax.lax.dot_general(x_ref[...], lw_ref[...], (((1,), (1,)), ((), ())),
                              preferred_element_type=jnp.float32)
    o_ref[...] = out + lb_ref[...]


def _pool_mat(n, out=8):
    m = np.zeros((n, out), dtype=np.float32)
    for i in range(out):
        h0 = (i * n) // out
        h1 = -((-(i + 1) * n) // out)
        m[h0:h1, i] = 1.0 / (h1 - h0)
    return m


def kernel(X, conv_w0, conv_b0, final_w0, final_b0, router_w0, router_b0, keys0,
           conv_w1, conv_b1, final_w1, final_b1, router_w1, router_b1, keys1,
           conv_w2, conv_b2, final_w2, final_b2, router_w2, router_b2, keys2,
           lin_w, lin_b):
    X = X.transpose(0, 2, 3, 1)
    X = jnp.pad(X, ((0, 0), (0, 0), (0, 0), (0, 5)))
    X = _moe_layer(X, 0, conv_w0, conv_b0, final_w0, final_b0, router_w0, router_b0, keys0)
    X = _moe_layer(X, 1, conv_w1, conv_b1, final_w1, final_b1, router_w1, router_b1, keys1)
    X = _moe_layer(X, 2, conv_w2, conv_b2, final_w2, final_b2, router_w2, router_b2, keys2)
    B, H, W, C = X.shape
    ncls = lin_w.shape[0]
    ph = jnp.asarray(_pool_mat(H).T)
    pw = _pool_mat(W)
    q_np = np.einsum('wj,cd->wcdj', pw, np.eye(C, dtype=np.float32)).reshape(W * C, C * 8)
    q = jnp.asarray(q_np)
    lwp = lin_w.reshape(ncls, C, 8, 8).transpose(0, 2, 1, 3).reshape(ncls, C * 64)
    pooled = pl.pallas_call(
        functools.partial(_pool_kernel, H, W, C),
        grid=(B,),
        in_specs=[
            pl.BlockSpec((1, H, W, C), lambda b: (b, 0, 0, 0)),
            pl.BlockSpec((8, H), lambda b: (0, 0)),
            pl.BlockSpec((W * C, C * 8), lambda b: (0, 0)),
        ],
        out_specs=pl.BlockSpec((1, 8, C * 8), lambda b: (b, 0, 0)),
        out_shape=jax.ShapeDtypeStruct((B, 8, C * 8), jnp.float32),
    )(X, ph, q)
    flat = pooled.reshape(B, C * 64)
    out = pl.pallas_call(
        _cls_kernel,
        in_specs=[
            pl.BlockSpec((B, C * 64), lambda: (0, 0)),
            pl.BlockSpec((ncls, C * 64), lambda: (0, 0)),
            pl.BlockSpec((1, ncls), lambda: (0, 0)),
        ],
        out_specs=pl.BlockSpec((B, ncls), lambda: (0, 0)),
        out_shape=jax.ShapeDtypeStruct((B, ncls), jnp.float32),
    )(flat, lwp, lin_b.reshape(1, ncls))
    return out


# speedup vs baseline: 17.1108x; 1.4377x over previous
import functools

import jax
import jax.numpy as jnp
import numpy as np
from jax.experimental import pallas as pl

_FF = 4
_LAYERS = [dict(cin=3, cout=8, ps=16), dict(cin=8, cout=16, ps=13), dict(cin=16, cout=16, ps=10)]
_E = 8


def _fourier_patch_means(H, W, hp, wp, ps):
    ys = ((np.arange(H, dtype=np.float32) + 0.5) / np.float32(H) * np.float32(2.0)
          - np.float32(1.0)).astype(np.float32)
    xs = ((np.arange(W, dtype=np.float32) + 0.5) / np.float32(W) * np.float32(2.0)
          - np.float32(1.0)).astype(np.float32)
    yy, xx = np.meshgrid(ys, xs, indexing='ij')
    feats = [xx, yy, (xx * yy).astype(np.float32),
             (xx.astype(np.float32) ** 2 + yy.astype(np.float32) ** 2).astype(np.float32)]
    for f in range(_FF):
        s = np.float32((2.0 ** f) * np.pi)
        for g in (xx, yy, (xx + yy).astype(np.float32), (xx - yy).astype(np.float32)):
            feats.append(np.sin(s * g, dtype=np.float32))
            feats.append(np.cos(s * g, dtype=np.float32))
    four = np.stack(feats, axis=0).astype(np.float32)
    fc = four.shape[0]
    fmean = four.reshape(fc, hp, ps, wp, ps).mean(axis=(2, 4), dtype=np.float32)
    return fmean.transpose(1, 2, 0).reshape(hp * wp, fc).astype(np.float32)


def _moe_kernel(ps, Win, Wv, Wp, C, cout, wp,
                x_ref, fb_ref, mx_ref, wc_ref, bc_ref, fwb_ref, fbb_ref,
                mneg_ref, mpos_ref, rep_ref, rept_ref, o_ref):
    EC = _E * cout
    x = x_ref[...].reshape(ps, Win, C)
    if Win != Wv:
        x = x[:, :Wv, :]
    if Wp != Wv:
        x = jnp.concatenate([x, jnp.zeros((ps, Wp - Wv, C), jnp.float32)], axis=1)
    xm = jnp.mean(x, axis=0)
    pm = jnp.dot(rept_ref[...], xm, preferred_element_type=jnp.float32) * (1.0 / ps)
    logits = jnp.dot(pm, mx_ref[...], preferred_element_type=jnp.float32)
    logits = logits + fb_ref[...].reshape(wp, _E)
    mmax = jnp.max(logits, axis=-1, keepdims=True)
    ex = jnp.exp(logits - mmax)
    scores = ex / jnp.sum(ex, axis=-1, keepdims=True)
    lane = jax.lax.broadcasted_iota(jnp.int32, (_E, EC), 1) // cout
    row = jax.lax.broadcasted_iota(jnp.int32, (_E, EC), 0)
    sel = (lane == row).astype(jnp.float32)
    sexp = jnp.dot(scores, sel, preferred_element_type=jnp.float32)
    sexp_w = jnp.dot(rep_ref[...], sexp, preferred_element_type=jnp.float32)
    z1 = jnp.zeros((ps, 1, C), jnp.float32)
    xm1 = jnp.concatenate([z1, x[:, :Wp - 1, :]], axis=1) * mneg_ref[...]
    xp1 = jnp.concatenate([x[:, 1:, :], z1], axis=1) * mpos_ref[...]
    a2 = jnp.concatenate([xm1, x, xp1], axis=2)
    zrow = jnp.zeros((1, Wp, 3 * C), jnp.float32)
    a2p = jnp.concatenate([zrow, a2, zrow], axis=0)
    z = jnp.zeros((ps * Wp, EC), jnp.float32) + bc_ref[...]
    for dy in range(3):
        opnd = a2p[dy:dy + ps].reshape(ps * Wp, 3 * C)
        z = z + jnp.dot(opnd, wc_ref[dy],
                        preferred_element_type=jnp.float32)
    z = jnp.maximum(z, 0.0).reshape(ps, Wp, EC) * sexp_w[None]
    out = jnp.dot(z.reshape(ps * Wp, EC), fwb_ref[...],
                  preferred_element_type=jnp.float32) + fbb_ref[...]
    out = out.reshape(ps, Wp, cout)
    if Wp != Wv:
        out = out[:, :Wv, :]
    o_ref[...] = out[None]


def _moe_layer(X, l, conv_w, conv_b, final_w, final_b, router_w, router_b, keys):
    cfg = _LAYERS[l]
    ps, cin, cout = cfg['ps'], cfg['cin'], cfg['cout']
    B, Hin, Win, C = X.shape
    hp, wp = Hin // ps, Win // ps
    Wv = wp * ps
    P = hp * wp
    EC = _E * cout
    fc = 4 + 8 * _FF
    Wr = router_w[:, :, 0, 0]
    M = Wr.T @ keys.T
    cvec = router_b @ keys.T
    fmean = jnp.asarray(_fourier_patch_means(hp * ps, Wv, hp, wp, ps))
    fbias = (fmean @ M[cin:] + cvec).reshape(hp, wp, _E)
    Mx = M[:cin]
    if C != cin:
        Mx = jnp.pad(Mx, ((0, C - cin), (0, 0)))
    Wc = conv_w.transpose(3, 4, 2, 0, 1)
    if C != cin:
        Wc = jnp.pad(Wc, ((0, 0), (0, 0), (0, C - cin), (0, 0), (0, 0)))
    Wc = Wc.reshape(3, 3 * C, EC)
    bc = conv_b.reshape(1, EC)
    FwB = jnp.tile(final_w[:, :, 0, 0].T, (_E, 1))
    fbb = final_b.reshape(1, cout)
    Wp = -((-Wv) // 8) * 8
    wl = np.arange(Wp) % ps
    valid = np.arange(Wp) < Wv
    mneg = jnp.asarray(((wl != 0) & valid).astype(np.float32).reshape(1, Wp, 1))
    mpos = jnp.asarray(((wl != ps - 1) & valid).astype(np.float32).reshape(1, Wp, 1))
    rep_np = np.zeros((Wp, wp), dtype=np.float32)
    rep_np[np.arange(Wv), np.arange(Wv) // ps] = 1.0
    rep = jnp.asarray(rep_np)
    rept = jnp.asarray(rep_np.T)
    kfn = functools.partial(_moe_kernel, ps, Win, Wv, Wp, C, cout, wp)
    out = pl.pallas_call(
        kfn,
        grid=(B, hp),
        in_specs=[
            pl.BlockSpec((1, ps, Win, C), lambda b, i: (b, i, 0, 0)),
            pl.BlockSpec((1, wp, _E), lambda b, i: (i, 0, 0)),
            pl.BlockSpec((C, _E), lambda b, i: (0, 0)),
            pl.BlockSpec((3, 3 * C, EC), lambda b, i: (0, 0, 0)),
            pl.BlockSpec((1, EC), lambda b, i: (0, 0)),
            pl.BlockSpec((EC, cout), lambda b, i: (0, 0)),
            pl.BlockSpec((1, cout), lambda b, i: (0, 0)),
            pl.BlockSpec((1, Wp, 1), lambda b, i: (0, 0, 0)),
            pl.BlockSpec((1, Wp, 1), lambda b, i: (0, 0, 0)),
            pl.BlockSpec((Wp, wp), lambda b, i: (0, 0)),
            pl.BlockSpec((wp, Wp), lambda b, i: (0, 0)),
        ],
        out_specs=pl.BlockSpec((1, ps, Wv, cout), lambda b, i: (b, i, 0, 0)),
        out_shape=jax.ShapeDtypeStruct((B, hp * ps, Wv, cout), jnp.float32),
    )(X, fbias, Mx, Wc, bc, FwB, fbb, mneg, mpos, rep, rept)
    return out


def _pool_kernel(H, W, C, x_ref, ph_ref, q_ref, o_ref):
    x = x_ref[...].reshape(H, W * C)
    t = jnp.dot(ph_ref[...], x, preferred_element_type=jnp.float32)
    s = jnp.dot(t, q_ref[...], preferred_element_type=jnp.float32)
    o_ref[...] = s[None]


def _cls_kernel(x_ref, lw_ref, lb_ref, o_ref):
    out = jax.lax.dot_general(x_ref[...], lw_ref[...], (((1,), (1,)), ((), ())),
                              preferred_element_type=jnp.float32)
    o_ref[...] = out + lb_ref[...]


def _pool_mat(n, out=8):
    m = np.zeros((n, out), dtype=np.float32)
    for i in range(out):
        h0 = (i * n) // out
        h1 = -((-(i + 1) * n) // out)
        m[h0:h1, i] = 1.0 / (h1 - h0)
    return m


def kernel(X, conv_w0, conv_b0, final_w0, final_b0, router_w0, router_b0, keys0,
           conv_w1, conv_b1, final_w1, final_b1, router_w1, router_b1, keys1,
           conv_w2, conv_b2, final_w2, final_b2, router_w2, router_b2, keys2,
           lin_w, lin_b):
    X = X.transpose(0, 2, 3, 1)
    X = jnp.pad(X, ((0, 0), (0, 0), (0, 0), (0, 5)))
    X = _moe_layer(X, 0, conv_w0, conv_b0, final_w0, final_b0, router_w0, router_b0, keys0)
    X = _moe_layer(X, 1, conv_w1, conv_b1, final_w1, final_b1, router_w1, router_b1, keys1)
    X = _moe_layer(X, 2, conv_w2, conv_b2, final_w2, final_b2, router_w2, router_b2, keys2)
    B, H, W, C = X.shape
    ncls = lin_w.shape[0]
    ph = jnp.asarray(_pool_mat(H).T)
    pw = _pool_mat(W)
    q_np = np.einsum('wj,cd->wcdj', pw, np.eye(C, dtype=np.float32)).reshape(W * C, C * 8)
    q = jnp.asarray(q_np)
    lwp = lin_w.reshape(ncls, C, 8, 8).transpose(0, 2, 1, 3).reshape(ncls, C * 64)
    pooled = pl.pallas_call(
        functools.partial(_pool_kernel, H, W, C),
        grid=(B,),
        in_specs=[
            pl.BlockSpec((1, H, 1, W * C), lambda b: (b, 0, 0, 0)),
            pl.BlockSpec((8, H), lambda b: (0, 0)),
            pl.BlockSpec((W * C, C * 8), lambda b: (0, 0)),
        ],
        out_specs=pl.BlockSpec((1, 8, C * 8), lambda b: (b, 0, 0)),
        out_shape=jax.ShapeDtypeStruct((B, 8, C * 8), jnp.float32),
    )(X.reshape(B, H, 1, W * C), ph, q)
    flat = pooled.reshape(B, C * 64)
    out = pl.pallas_call(
        _cls_kernel,
        in_specs=[
            pl.BlockSpec((B, C * 64), lambda: (0, 0)),
            pl.BlockSpec((ncls, C * 64), lambda: (0, 0)),
            pl.BlockSpec((1, ncls), lambda: (0, 0)),
        ],
        out_specs=pl.BlockSpec((B, ncls), lambda: (0, 0)),
        out_shape=jax.ShapeDtypeStruct((B, ncls), jnp.float32),
    )(flat, lwp, lin_b.reshape(1, ncls))
    return out
